# baseline (speedup 1.0000x reference)
import functools

import jax
import jax.numpy as jnp
from jax import lax
from jax.experimental import pallas as pl
from jax.experimental.pallas import tpu as pltpu
from jax.experimental.pallas import tpu_sc as plsc

_TM = 256
_TC = 512


def _dist_argmin_body(x_ref, et_ref, idx_ref, loss_ref, enorm_ref, acc_ref):
    m = pl.program_id(0)
    nm = pl.num_programs(0)
    K = et_ref.shape[1]

    @pl.when(m == 0)
    def _init():
        et = et_ref[...]
        enorm_ref[...] = jnp.sum(et * et, axis=0, keepdims=True)
        acc_ref[0, 0] = 0.0

    x = x_ref[...]
    xnorm = jnp.sum(x * x, axis=1, keepdims=True)

    run_min = None
    run_idx = None
    for c in range(K // _TC):
        et_c = et_ref[:, c * _TC:(c + 1) * _TC]
        mm = jnp.dot(x, et_c, preferred_element_type=jnp.float32)
        scores = (xnorm - 2.0 * mm) + enorm_ref[:, c * _TC:(c + 1) * _TC]
        cmin = jnp.min(scores, axis=1, keepdims=True)
        lane = lax.broadcasted_iota(jnp.int32, scores.shape, 1) + c * _TC
        cand = jnp.where(scores == cmin, lane, jnp.int32(2**31 - 1))
        carg = jnp.min(cand, axis=1, keepdims=True)
        if run_min is None:
            run_min, run_idx = cmin, carg
        else:
            better = cmin < run_min
            run_min = jnp.where(better, cmin, run_min)
            run_idx = jnp.where(better, carg, run_idx)

    idx_ref[...] = run_idx
    acc_ref[0, 0] = acc_ref[0, 0] + jnp.sum(run_min)

    @pl.when(m == nm - 1)
    def _fin():
        loss_ref[0, 0] = acc_ref[0, 0]


def _dist_argmin(x, et):
    M, D = x.shape
    K = et.shape[1]
    return pl.pallas_call(
        _dist_argmin_body,
        grid=(M // _TM,),
        in_specs=[
            pl.BlockSpec((_TM, D), lambda m: (m, 0)),
            pl.BlockSpec((D, K), lambda m: (0, 0)),
        ],
        out_specs=[
            pl.BlockSpec((_TM, 1), lambda m: (m, 0)),
            pl.BlockSpec((1, 1), lambda m: (0, 0), memory_space=pltpu.SMEM),
        ],
        out_shape=[
            jax.ShapeDtypeStruct((M, 1), jnp.int32),
            jax.ShapeDtypeStruct((1, 1), jnp.float32),
        ],
        scratch_shapes=[
            pltpu.VMEM((1, K), jnp.float32),
            pltpu.SMEM((1, 1), jnp.float32),
        ],
    )(x, et)


_CH = 128


@functools.lru_cache(maxsize=None)
def _make_sc_gather(K, D, M):
    info = plsc.get_sparse_core_info()
    nw = info.num_cores * info.num_subcores
    rows_per_w = M // nw
    nch = rows_per_w // _CH
    mesh = plsc.VectorSubcoreMesh(core_axis_name="c", subcore_axis_name="s")

    @functools.partial(
        pl.kernel,
        mesh=mesh,
        out_type=jax.ShapeDtypeStruct((M, D), jnp.float32),
        scratch_types=[
            pltpu.VMEM((nch, _CH), jnp.int32),
            pltpu.VMEM((_CH, D), jnp.float32),
            pltpu.VMEM((_CH, D), jnp.float32),
            pltpu.SemaphoreType.DMA,
            pltpu.SemaphoreType.DMA,
        ],
    )
    def gather(table_hbm, idx_hbm, out_hbm, idx_v, rows_a, rows_b, sem_a, sem_b):
        wid = lax.axis_index("s") * info.num_cores + lax.axis_index("c")
        base = wid * rows_per_w
        pltpu.sync_copy(idx_hbm.at[pl.ds(wid * nch, nch)], idx_v)
        bufs = (rows_a, rows_b)
        sems = (sem_a, sem_b)
        h = pltpu.async_copy(table_hbm.at[idx_v.at[0]], bufs[0], sems[0])
        for ci in range(nch):
            h.wait()
            if ci + 1 < nch:
                h = pltpu.async_copy(
                    table_hbm.at[idx_v.at[ci + 1]],
                    bufs[(ci + 1) % 2], sems[(ci + 1) % 2])
            pltpu.sync_copy(bufs[ci % 2],
                            out_hbm.at[pl.ds(base + ci * _CH, _CH)])

    return gather


def kernel(z_real, z_imag, embedding):
    B, L, D = z_real.shape
    K = embedding.shape[0]
    M = B * L

    x = z_real.reshape(M, D)
    et = embedding.T
    idx2d, loss11 = _dist_argmin(x, et)

    idx = idx2d.reshape(M // _CH, _CH)
    zq_flat = _make_sc_gather(K, D, M)(embedding, idx)
    z_quant_real = zq_flat.reshape(B, L, D)

    vq_loss = (loss11 * (1.25 / (M * D))).reshape(())
    return z_quant_real, z_imag, vq_loss


# speedup vs baseline: 1.0256x; 1.0256x over previous
import functools

import jax
import jax.numpy as jnp
from jax import lax
from jax.experimental import pallas as pl
from jax.experimental.pallas import tpu as pltpu
from jax.experimental.pallas import tpu_sc as plsc

_TM = 256
_TC = 512


def _dist_argmin_body(x_ref, etn_ref, idx_ref, loss_ref, enorm_ref, acc_ref):
    m = pl.program_id(0)
    nm = pl.num_programs(0)
    K = etn_ref.shape[1]

    @pl.when(m == 0)
    def _init():
        etn = etn_ref[...]
        enorm_ref[...] = 0.25 * jnp.sum(etn * etn, axis=0, keepdims=True)
        acc_ref[0, 0] = 0.0

    x = x_ref[...]
    xnorm = jnp.sum(x * x, axis=1, keepdims=True)

    run_min = None
    run_idx = None
    for c in range(K // _TC):
        etn_c = etn_ref[:, c * _TC:(c + 1) * _TC]
        mm2 = jnp.dot(x, etn_c, preferred_element_type=jnp.float32)
        scores = mm2 + enorm_ref[:, c * _TC:(c + 1) * _TC]
        cmin = jnp.min(scores, axis=1, keepdims=True)
        lane = lax.broadcasted_iota(jnp.int32, scores.shape, 1) + c * _TC
        cand = jnp.where(scores == cmin, lane, jnp.int32(2**31 - 1))
        carg = jnp.min(cand, axis=1, keepdims=True)
        if run_min is None:
            run_min, run_idx = cmin, carg
        else:
            better = cmin < run_min
            run_min = jnp.where(better, cmin, run_min)
            run_idx = jnp.where(better, carg, run_idx)

    idx_ref[...] = run_idx
    acc_ref[0, 0] = acc_ref[0, 0] + jnp.sum(run_min + xnorm)

    @pl.when(m == nm - 1)
    def _fin():
        loss_ref[0, 0] = acc_ref[0, 0]


def _dist_argmin(x, et):
    M, D = x.shape
    K = et.shape[1]
    return pl.pallas_call(
        _dist_argmin_body,
        grid=(M // _TM,),
        in_specs=[
            pl.BlockSpec((_TM, D), lambda m: (m, 0)),
            pl.BlockSpec((D, K), lambda m: (0, 0)),
        ],
        out_specs=[
            pl.BlockSpec((_TM, 1), lambda m: (m, 0)),
            pl.BlockSpec((1, 1), lambda m: (0, 0), memory_space=pltpu.SMEM),
        ],
        out_shape=[
            jax.ShapeDtypeStruct((M, 1), jnp.int32),
            jax.ShapeDtypeStruct((1, 1), jnp.float32),
        ],
        scratch_shapes=[
            pltpu.VMEM((1, K), jnp.float32),
            pltpu.SMEM((1, 1), jnp.float32),
        ],
    )(x, et)


_CH = 128


@functools.lru_cache(maxsize=None)
def _make_sc_gather(K, D, M):
    info = plsc.get_sparse_core_info()
    nw = info.num_cores * info.num_subcores
    rows_per_w = M // nw
    nch = rows_per_w // _CH
    mesh = plsc.VectorSubcoreMesh(core_axis_name="c", subcore_axis_name="s")

    @functools.partial(
        pl.kernel,
        mesh=mesh,
        out_type=jax.ShapeDtypeStruct((M, D), jnp.float32),
        scratch_types=[
            pltpu.VMEM((nch, _CH), jnp.int32),
            pltpu.VMEM((_CH, D), jnp.float32),
            pltpu.VMEM((_CH, D), jnp.float32),
            pltpu.SemaphoreType.DMA,
            pltpu.SemaphoreType.DMA,
        ],
    )
    def gather(table_hbm, idx_hbm, out_hbm, idx_v, rows_a, rows_b, sem_a, sem_b):
        wid = lax.axis_index("s") * info.num_cores + lax.axis_index("c")
        base = wid * rows_per_w
        pltpu.sync_copy(idx_hbm.at[pl.ds(wid * nch, nch)], idx_v)
        bufs = (rows_a, rows_b)
        sems = (sem_a, sem_b)
        h = pltpu.async_copy(table_hbm.at[idx_v.at[0]], bufs[0], sems[0])
        for ci in range(nch):
            h.wait()
            if ci + 1 < nch:
                h = pltpu.async_copy(
                    table_hbm.at[idx_v.at[ci + 1]],
                    bufs[(ci + 1) % 2], sems[(ci + 1) % 2])
            pltpu.sync_copy(bufs[ci % 2],
                            out_hbm.at[pl.ds(base + ci * _CH, _CH)])

    return gather


def kernel(z_real, z_imag, embedding):
    B, L, D = z_real.shape
    K = embedding.shape[0]
    M = B * L

    x = z_real.reshape(M, D)
    etn = embedding.T * (-2.0)
    idx2d, loss11 = _dist_argmin(x, etn)

    idx = idx2d.reshape(M // _CH, _CH)
    zq_flat = _make_sc_gather(K, D, M)(embedding, idx)
    z_quant_real = zq_flat.reshape(B, L, D)

    vq_loss = (loss11 * (1.25 / (M * D))).reshape(())
    return z_quant_real, z_imag, vq_loss
